# Initial kernel scaffold; baseline (speedup 1.0000x reference)
import jax
import jax.numpy as jnp
from jax.experimental import pallas as pl


def kernel(x, pe):
    raise NotImplementedError("write your pallas kernel here")


# speedup vs baseline: 1.6648x; 1.6648x over previous
import functools

import jax
import jax.numpy as jnp
from jax import lax
from jax.experimental import pallas as pl
from jax.experimental.pallas import tpu as pltpu
from jax.experimental.pallas import tpu_sc as plsc

NC, NS = 2, 16
NW = NC * NS
BATCH, SEQ = 4, 2048
N_IDX = BATCH * SEQ
D = 1024
ROWS_PER_W = N_IDX // NW
CH = 32
NCHUNK = ROWS_PER_W // CH
NBUF = 3

_mesh = plsc.VectorSubcoreMesh(core_axis_name="c", subcore_axis_name="s")


@functools.partial(
    pl.kernel,
    mesh=_mesh,
    out_type=jax.ShapeDtypeStruct((N_IDX, D), jnp.float32),
    scratch_types=[
        pltpu.VMEM((NCHUNK, CH), jnp.int32),
        pltpu.VMEM((CH, D), jnp.float32),
        pltpu.VMEM((CH, D), jnp.float32),
        pltpu.VMEM((CH, D), jnp.float32),
        pltpu.SemaphoreType.DMA,
        pltpu.SemaphoreType.DMA,
    ],
)
def _gather_kernel(x_hbm, pe_hbm, out_hbm, idx_v, buf0, buf1, buf2, gsem, osem):
    wid = lax.axis_index("s") * NC + lax.axis_index("c")
    base = wid * ROWS_PER_W
    bufs = (buf0, buf1, buf2)

    pltpu.sync_copy(x_hbm.at[wid], idx_v)

    def gather(c):
        return pltpu.async_copy(pe_hbm.at[idx_v.at[c]], bufs[c % NBUF], gsem)

    gathers = [None] * NCHUNK
    outs = [None] * NCHUNK
    gathers[0] = gather(0)
    gathers[1] = gather(1)
    waited_out = 0
    for c in range(NCHUNK):
        gathers[c].wait()
        outs[c] = pltpu.async_copy(
            bufs[c % NBUF], out_hbm.at[pl.ds(base + c * CH, CH)], osem
        )
        if c + 2 < NCHUNK:
            if c >= 1:
                outs[waited_out].wait()
                waited_out += 1
            gathers[c + 2] = gather(c + 2)
    for i in range(waited_out, NCHUNK):
        outs[i].wait()


def kernel(x, pe):
    xr = x.reshape(NW, NCHUNK, CH)
    out = _gather_kernel(xr, pe)
    return out.reshape(BATCH, SEQ, D)


# speedup vs baseline: 1.6661x; 1.0008x over previous
import functools

import jax
import jax.numpy as jnp
from jax import lax
from jax.experimental import pallas as pl
from jax.experimental.pallas import tpu as pltpu
from jax.experimental.pallas import tpu_sc as plsc

NC, NS = 2, 16
NW = NC * NS
BATCH, SEQ = 4, 2048
N_IDX = BATCH * SEQ
D = 1024
ROWS_PER_W = N_IDX // NW
CH = 16
NCHUNK = ROWS_PER_W // CH
NBUF = 7

_mesh = plsc.VectorSubcoreMesh(core_axis_name="c", subcore_axis_name="s")


@functools.partial(
    pl.kernel,
    mesh=_mesh,
    out_type=jax.ShapeDtypeStruct((N_IDX, D), jnp.float32),
    scratch_types=[
        pltpu.VMEM((NCHUNK, CH), jnp.int32),
    ]
    + [pltpu.VMEM((CH, D), jnp.float32) for _ in range(NBUF)]
    + [
        pltpu.SemaphoreType.DMA,
        pltpu.SemaphoreType.DMA,
    ],
)
def _gather_kernel(x_hbm, pe_hbm, out_hbm, idx_v, *rest):
    bufs = rest[:NBUF]
    gsem, osem = rest[NBUF], rest[NBUF + 1]
    wid = lax.axis_index("s") * NC + lax.axis_index("c")
    base = wid * ROWS_PER_W

    pltpu.sync_copy(x_hbm.at[wid], idx_v)

    def gather(c):
        return pltpu.async_copy(pe_hbm.at[idx_v.at[c]], bufs[c % NBUF], gsem)

    gathers = [None] * NCHUNK
    outs = [None] * NCHUNK
    for c in range(NBUF - 1):
        gathers[c] = gather(c)
    waited_out = 0
    for c in range(NCHUNK):
        gathers[c].wait()
        outs[c] = pltpu.async_copy(
            bufs[c % NBUF], out_hbm.at[pl.ds(base + c * CH, CH)], osem
        )
        nxt = c + NBUF - 1
        if nxt < NCHUNK:
            if nxt - NBUF >= 0:
                outs[waited_out].wait()
                waited_out += 1
            gathers[nxt] = gather(nxt)
    for i in range(waited_out, NCHUNK):
        outs[i].wait()


def kernel(x, pe):
    xr = x.reshape(NW, NCHUNK, CH)
    out = _gather_kernel(xr, pe)
    return out.reshape(BATCH, SEQ, D)


# speedup vs baseline: 1.6678x; 1.0010x over previous
import functools

import jax
import jax.numpy as jnp
from jax import lax
from jax.experimental import pallas as pl
from jax.experimental.pallas import tpu as pltpu
from jax.experimental.pallas import tpu_sc as plsc

NC, NS = 2, 16
NW = NC * NS
BATCH, SEQ = 4, 2048
N_IDX = BATCH * SEQ
D = 1024
ROWS_PER_W = N_IDX // NW
W_PER_ROW = SEQ // ROWS_PER_W
CH = 16
NCHUNK = ROWS_PER_W // CH
NBUF = 7

_mesh = plsc.VectorSubcoreMesh(core_axis_name="c", subcore_axis_name="s")


@functools.partial(
    pl.kernel,
    mesh=_mesh,
    out_type=jax.ShapeDtypeStruct((N_IDX, D), jnp.float32),
    scratch_types=[
        pltpu.VMEM((ROWS_PER_W,), jnp.int32),
    ]
    + [pltpu.VMEM((CH, D), jnp.float32) for _ in range(NBUF)]
    + [
        pltpu.SemaphoreType.DMA,
        pltpu.SemaphoreType.DMA,
    ],
)
def _gather_kernel(x_hbm, pe_hbm, out_hbm, idx_v, *rest):
    bufs = rest[:NBUF]
    gsem, osem = rest[NBUF], rest[NBUF + 1]
    wid = lax.axis_index("s") * NC + lax.axis_index("c")
    base = wid * ROWS_PER_W

    pltpu.sync_copy(
        x_hbm.at[wid // W_PER_ROW, pl.ds((wid % W_PER_ROW) * ROWS_PER_W, ROWS_PER_W)],
        idx_v,
    )

    def gather(c):
        return pltpu.async_copy(
            pe_hbm.at[idx_v.at[pl.ds(c * CH, CH)]], bufs[c % NBUF], gsem
        )

    gathers = [None] * NCHUNK
    outs = [None] * NCHUNK
    for c in range(NBUF - 1):
        gathers[c] = gather(c)
    waited_out = 0
    for c in range(NCHUNK):
        gathers[c].wait()
        outs[c] = pltpu.async_copy(
            bufs[c % NBUF], out_hbm.at[pl.ds(base + c * CH, CH)], osem
        )
        nxt = c + NBUF - 1
        if nxt < NCHUNK:
            if nxt - NBUF >= 0:
                outs[waited_out].wait()
                waited_out += 1
            gathers[nxt] = gather(nxt)
    for i in range(waited_out, NCHUNK):
        outs[i].wait()


def kernel(x, pe):
    out = _gather_kernel(x, pe)
    return out.reshape(BATCH, SEQ, D)


# speedup vs baseline: 1.6838x; 1.0096x over previous
import functools

import jax
import jax.numpy as jnp
from jax import lax
from jax.experimental import pallas as pl
from jax.experimental.pallas import tpu as pltpu
from jax.experimental.pallas import tpu_sc as plsc

NC, NS = 2, 16
NW = NC * NS
BATCH, SEQ = 4, 2048
N_IDX = BATCH * SEQ
D = 1024
ROWS_PER_W = N_IDX // NW
W_PER_ROW = SEQ // ROWS_PER_W
CH = 16
NCHUNK = ROWS_PER_W // CH
NBUF = 4

_mesh = plsc.VectorSubcoreMesh(core_axis_name="c", subcore_axis_name="s")


@functools.partial(
    pl.kernel,
    mesh=_mesh,
    out_type=jax.ShapeDtypeStruct((N_IDX, D), jnp.float32),
    scratch_types=[
        pltpu.VMEM((ROWS_PER_W,), jnp.int32),
    ]
    + [pltpu.VMEM((CH, D), jnp.float32) for _ in range(NBUF)]
    + [
        pltpu.SemaphoreType.DMA,
        pltpu.SemaphoreType.DMA,
    ],
)
def _gather_kernel(x_hbm, pe_hbm, out_hbm, idx_v, *rest):
    bufs = rest[:NBUF]
    gsem, osem = rest[NBUF], rest[NBUF + 1]
    wid = lax.axis_index("s") * NC + lax.axis_index("c")
    base = wid * ROWS_PER_W

    pltpu.sync_copy(
        x_hbm.at[wid // W_PER_ROW, pl.ds((wid % W_PER_ROW) * ROWS_PER_W, ROWS_PER_W)],
        idx_v,
    )

    def gather(c, buf):
        pltpu.async_copy(pe_hbm.at[idx_v.at[pl.ds(c * CH, CH)]], buf, gsem)

    for b in range(NBUF):
        gather(b, bufs[b])

    @pl.loop(0, NCHUNK, step=NBUF)
    def _chunks(i):
        for b in range(NBUF):
            c = i + b
            pltpu.make_async_copy(pe_hbm.at[pl.ds(0, CH)], bufs[b], gsem).wait()
            out_cp = pltpu.async_copy(
                bufs[b], out_hbm.at[pl.ds(base + c * CH, CH)], osem
            )
            out_cp.wait()

            @pl.when(c + NBUF < NCHUNK)
            def _():
                gather(c + NBUF, bufs[b])


def kernel(x, pe):
    out = _gather_kernel(x, pe)
    return out.reshape(BATCH, SEQ, D)
